# Initial kernel scaffold; baseline (speedup 1.0000x reference)
import jax
import jax.numpy as jnp
from jax.experimental import pallas as pl


def kernel(x, edge_index, edge_attr, eps, B0, B1, B2, W1, b1, ln_g, ln_b, W2, b2):
    raise NotImplementedError("write your pallas kernel here")


# speedup vs baseline: 1.9885x; 1.9885x over previous
import functools

import jax
import jax.numpy as jnp
from jax import lax
from jax.experimental import pallas as pl
from jax.experimental.pallas import tpu as pltpu
from jax.experimental.pallas import tpu_sc as plsc

_N = 10000
_D = 128
_E = 320000

_NC = 2
_NS = 16
_NW = _NC * _NS

_CHUNK = 128
_NCH = -(-_E // (_NW * _CHUNK)) * _NW
_EP = _NCH * _CHUNK
_CPT = _NCH // _NW
_NROWS = 10112
_RPT = _NROWS // _NS

_LANES = 16
_VPR = _D // _LANES


def _sc_body(x_hbm, tt_hbm, ea_hbm, srcm_hbm, dstm_hbm, out_hbm,
             srcv, dstv, eav, cv, xbuf, tbuf, sem1, sem2, acc):
    cid = lax.axis_index("c")
    sid = lax.axis_index("s")
    wid = sid * _NC + cid

    def _zero_row(i, _):
        for j in range(_VPR):
            xbuf[i, pl.ds(j * _LANES, _LANES)] = jnp.zeros((_LANES,), jnp.float32)
        return _
    lax.fori_loop(0, _CHUNK, _zero_row, None)

    base = sid * _RPT
    full = _RPT // _CHUNK
    for k in range(full):
        pltpu.sync_copy(xbuf, acc.at[pl.ds(base + k * _CHUNK, _CHUNK)])
    rem = _RPT - full * _CHUNK
    if rem:
        pltpu.sync_copy(xbuf.at[pl.ds(0, rem)], acc.at[pl.ds(base + full * _CHUNK, rem)])
    plsc.subcore_barrier()

    def _chunk(k, _):
        row = wid * _CPT + k
        pltpu.sync_copy(srcm_hbm.at[row], srcv)
        pltpu.sync_copy(dstm_hbm.at[row], dstv)
        pltpu.sync_copy(ea_hbm.at[row], eav)
        for j in range(_CHUNK // _LANES):
            sl = pl.ds(j * _LANES, _LANES)
            cv[sl] = eav[0, sl] * 12 + eav[1, sl] * 2 + eav[2, sl]
        cp1 = pltpu.async_copy(tt_hbm.at[cv], tbuf, sem1)
        cp2 = pltpu.async_copy(x_hbm.at[srcv], xbuf, sem2)
        cp1.wait()
        cp2.wait()

        def _msg_row(i, _c):
            for j in range(_VPR):
                sl = pl.ds(j * _LANES, _LANES)
                xbuf[i, sl] = jnp.maximum(xbuf[i, sl] + tbuf[i, sl], 0.0)
            return _c
        lax.fori_loop(0, _CHUNK, _msg_row, None)

        pltpu.sync_copy(xbuf, acc.at[dstv], add=True)
        return _

    lax.fori_loop(0, _CPT, _chunk, None)
    plsc.subcore_barrier()

    pltpu.sync_copy(acc.at[pl.ds(base, _RPT)], out_hbm.at[cid, pl.ds(base, _RPT)])


_sc_aggregate = pl.kernel(
    _sc_body,
    out_type=jax.ShapeDtypeStruct((_NC, _NROWS, _D), jnp.float32),
    mesh=plsc.VectorSubcoreMesh(
        core_axis_name="c", subcore_axis_name="s",
        num_cores=_NC, num_subcores=_NS),
    scratch_types=[
        pltpu.VMEM((_CHUNK,), jnp.int32),
        pltpu.VMEM((_CHUNK,), jnp.int32),
        pltpu.VMEM((3, _CHUNK), jnp.int32),
        pltpu.VMEM((_CHUNK,), jnp.int32),
        pltpu.VMEM((_CHUNK, _D), jnp.float32),
        pltpu.VMEM((_CHUNK, _D), jnp.float32),
        pltpu.SemaphoreType.DMA,
        pltpu.SemaphoreType.DMA,
        pltpu.VMEM_SHARED((_NROWS, _D), jnp.float32),
    ],
)


def _tc_body(scale_ref, x_ref, p_ref, w1_ref, b1_ref, g_ref, bb_ref,
             w2_ref, b2_ref, o_ref):
    h = scale_ref[0, 0] * x_ref[...] + p_ref[0] + p_ref[1]
    h = jnp.dot(h, w1_ref[...], preferred_element_type=jnp.float32) + b1_ref[...]
    mu = jnp.mean(h, axis=-1, keepdims=True)
    var = jnp.mean((h - mu) * (h - mu), axis=-1, keepdims=True)
    h = (h - mu) * lax.rsqrt(var + 1e-5) * g_ref[...] + bb_ref[...]
    h = jnp.maximum(h, 0.0)
    o_ref[...] = jnp.dot(h, w2_ref[...], preferred_element_type=jnp.float32) + b2_ref[...]


_BR = 1000


def _tc_mlp(scale, x, partials, W1, b1, ln_g, ln_b, W2, b2):
    grid = (_N // _BR,)
    return pl.pallas_call(
        _tc_body,
        grid=grid,
        in_specs=[
            pl.BlockSpec((1, 1), lambda i: (0, 0), memory_space=pltpu.SMEM),
            pl.BlockSpec((_BR, _D), lambda i: (i, 0)),
            pl.BlockSpec((_NC, _BR, _D), lambda i: (0, i, 0)),
            pl.BlockSpec((_D, _D), lambda i: (0, 0)),
            pl.BlockSpec((1, _D), lambda i: (0, 0)),
            pl.BlockSpec((1, _D), lambda i: (0, 0)),
            pl.BlockSpec((1, _D), lambda i: (0, 0)),
            pl.BlockSpec((_D, _D), lambda i: (0, 0)),
            pl.BlockSpec((1, _D), lambda i: (0, 0)),
        ],
        out_specs=pl.BlockSpec((_BR, _D), lambda i: (i, 0)),
        out_shape=jax.ShapeDtypeStruct((_N, _D), jnp.float32),
    )(scale, x, partials, W1, b1, ln_g, ln_b, W2, b2)


def kernel(x, edge_index, edge_attr, eps, B0, B1, B2, W1, b1, ln_g, ln_b, W2, b2):
    tt = (B0[:, None, None, :] + B1[None, :, None, :] + B2[None, None, :, :]
          ).reshape(5 * 6 * 2, _D)

    pad = _EP - _E
    src = jnp.concatenate([edge_index[0], jnp.zeros((pad,), jnp.int32)])
    dst = jnp.concatenate([edge_index[1], jnp.full((pad,), _N, jnp.int32)])
    ea = jnp.concatenate([edge_attr, jnp.zeros((pad, 3), jnp.int32)])
    srcm = src.reshape(_NCH, _CHUNK)
    dstm = dst.reshape(_NCH, _CHUNK)
    eam = ea.reshape(_NCH, _CHUNK, 3).transpose(0, 2, 1)

    partials = _sc_aggregate(x, tt, eam, srcm, dstm)

    scale = (1.0 + eps).reshape(1, 1)
    return _tc_mlp(scale, x, partials, W1, b1.reshape(1, _D),
                   ln_g.reshape(1, _D), ln_b.reshape(1, _D), W2, b2.reshape(1, _D))
